# Initial kernel scaffold; baseline (speedup 1.0000x reference)
import jax
import jax.numpy as jnp
from jax.experimental import pallas as pl


def kernel(users, items, users_ratings, items_ratings, users_sparse_ratings, items_sparse_ratings, UsersEmb, ItemsEmb, UsersRatingsEmb, ItemsRatingsEmb, W_ui, W_ur, W_ri, W_rr):
    raise NotImplementedError("write your pallas kernel here")


# speedup vs baseline: 4.7697x; 4.7697x over previous
import functools

import jax
import jax.numpy as jnp
from jax import lax
from jax.experimental import pallas as pl
from jax.experimental.pallas import tpu as pltpu
from jax.experimental.pallas import tpu_sc as plsc

B = 1024
D = 64
H = 200
NC = 2
NS = 16
NW = NC * NS
BPW = B // NW
H0 = 104
H1 = H - H0
NLANE = 16
DV = D // NLANE


def _sc_gather_pool(users_r, items_r, ur_idx_r, ir_idx_r,
                    uemb_r, iemb_r, uremb_r, iremb_r,
                    ue_out, ie_out, ure_out, ire_out,
                    idx_v, rows_v, hidx_v, hrows0_v, hrows1_v, pooled_v, sem):
    wid = lax.axis_index("s") * NC + lax.axis_index("c")
    base = wid * BPW

    def gather_simple(src_idx_hbm, table_hbm, out_hbm):
        pltpu.sync_copy(src_idx_hbm.at[pl.ds(base, BPW)], idx_v)
        pltpu.async_copy(table_hbm.at[idx_v], rows_v, sem).wait()
        pltpu.sync_copy(rows_v, out_hbm.at[pl.ds(base, BPW)])

    gather_simple(users_r, uemb_r, ue_out)
    gather_simple(items_r, iemb_r, ie_out)

    def pool_table(hist_idx_hbm, table_hbm, out_hbm):
        pltpu.sync_copy(hist_idx_hbm.at[pl.ds(base * H, BPW * H)], hidx_v)
        bufs = (hrows0_v, hrows1_v)

        def fire(b):
            buf = bufs[b % 2]
            c0 = pltpu.async_copy(
                table_hbm.at[hidx_v.at[pl.ds(b * H, H0)]],
                buf.at[pl.ds(0, H0)], sem)
            c1 = pltpu.async_copy(
                table_hbm.at[hidx_v.at[pl.ds(b * H + H0, H1)]],
                buf.at[pl.ds(H0, H1)], sem)
            return (c0, c1)

        inflight = fire(0)
        for b in range(BPW):
            for c in inflight:
                c.wait()
            buf = bufs[b % 2]
            if b + 1 < BPW:
                inflight = fire(b + 1)

            def body(j, acc):
                return tuple(
                    acc[k] + buf[j, pl.ds(k * NLANE, NLANE)]
                    for k in range(DV))

            acc0 = tuple(jnp.zeros((NLANE,), jnp.float32) for _ in range(DV))
            acc = lax.fori_loop(0, H, body, acc0)
            scale = jnp.float32(1.0 / H)
            for k in range(DV):
                pooled_v[b, pl.ds(k * NLANE, NLANE)] = acc[k] * scale
        pltpu.sync_copy(pooled_v, out_hbm.at[pl.ds(base, BPW)])

    pool_table(ur_idx_r, uremb_r, ure_out)
    pool_table(ir_idx_r, iremb_r, ire_out)


@functools.partial(
    pl.kernel,
    out_type=tuple(jax.ShapeDtypeStruct((B, D), jnp.float32) for _ in range(4)),
    mesh=plsc.VectorSubcoreMesh(core_axis_name="c", subcore_axis_name="s"),
    scratch_types=[
        pltpu.VMEM((BPW,), jnp.int32),
        pltpu.VMEM((BPW, D), jnp.float32),
        pltpu.VMEM((BPW * H,), jnp.int32),
        pltpu.VMEM((H, D), jnp.float32),
        pltpu.VMEM((H, D), jnp.float32),
        pltpu.VMEM((BPW, D), jnp.float32),
        pltpu.SemaphoreType.DMA,
    ],
    compiler_params=pltpu.CompilerParams(use_tc_tiling_on_sc=False),
)
def _sc_kernel(*refs):
    _sc_gather_pool(*refs)


def _tc_combine(ue_ref, ie_ref, ure_ref, ire_ref,
                wui_ref, wur_ref, wri_ref, wrr_ref,
                total_ref, regs_ref):
    ue = ue_ref[...]
    ie = ie_ref[...]
    ure = ure_ref[...]
    ire = ire_ref[...]
    inf_ui = jnp.dot(ue * ie, wui_ref[...], preferred_element_type=jnp.float32)
    inf_ur = jnp.dot(ue + ire, wur_ref[...], preferred_element_type=jnp.float32)
    inf_ri = jnp.dot(jnp.maximum(ure, ie), wri_ref[...],
                     preferred_element_type=jnp.float32)
    inf_rr = (jnp.dot(ure, wrr_ref[0:D, :], preferred_element_type=jnp.float32)
              + jnp.dot(ire, wrr_ref[D:2 * D, :],
                        preferred_element_type=jnp.float32))
    total_ref[...] = 0.25 * (inf_ui + inf_ur + inf_ri + inf_rr)
    regs = 0.001 * (jnp.sqrt(jnp.sum(ue * ue)) + jnp.sqrt(jnp.sum(ie * ie)))
    regs_ref[...] = regs.reshape(1, 1)


_tc_combine_call = pl.pallas_call(
    _tc_combine,
    out_shape=(jax.ShapeDtypeStruct((B, 1), jnp.float32),
               jax.ShapeDtypeStruct((1, 1), jnp.float32)),
)


def kernel(users, items, users_ratings, items_ratings,
           users_sparse_ratings, items_sparse_ratings,
           UsersEmb, ItemsEmb, UsersRatingsEmb, ItemsRatingsEmb,
           W_ui, W_ur, W_ri, W_rr):
    del users_sparse_ratings, items_sparse_ratings
    users = users.astype(jnp.int32)
    items = items.astype(jnp.int32)
    users_ratings = users_ratings.astype(jnp.int32).reshape(B * H)
    items_ratings = items_ratings.astype(jnp.int32).reshape(B * H)
    ue, ie, ure, ire = _sc_kernel(
        users, items, users_ratings, items_ratings,
        UsersEmb, ItemsEmb, UsersRatingsEmb, ItemsRatingsEmb)
    total, regs = _tc_combine_call(ue, ie, ure, ire, W_ui, W_ur, W_ri, W_rr)
    return total, regs[0, 0]


# speedup vs baseline: 4.7704x; 1.0001x over previous
import functools

import jax
import jax.numpy as jnp
from jax import lax
from jax.experimental import pallas as pl
from jax.experimental.pallas import tpu as pltpu
from jax.experimental.pallas import tpu_sc as plsc

B = 1024
D = 64
H = 200
NC = 2
NS = 16
NW = NC * NS
BPW = B // NW
H0 = 104
H1 = H - H0
NLANE = 16
DV = D // NLANE


def _sc_gather_pool(users_r, items_r, ur_idx_r, ir_idx_r,
                    uemb_r, iemb_r, uremb_r, iremb_r,
                    ue_out, ie_out, ure_out, ire_out,
                    idx_v, rows_v, hidx_v, hrows0_v, hrows1_v, pooled_v, sem):
    wid = lax.axis_index("s") * NC + lax.axis_index("c")
    base = wid * BPW

    def gather_simple(src_idx_hbm, table_hbm, out_hbm):
        pltpu.sync_copy(src_idx_hbm.at[pl.ds(base, BPW)], idx_v)
        pltpu.async_copy(table_hbm.at[idx_v], rows_v, sem).wait()
        pltpu.sync_copy(rows_v, out_hbm.at[pl.ds(base, BPW)])

    gather_simple(users_r, uemb_r, ue_out)
    gather_simple(items_r, iemb_r, ie_out)

    def pool_table(hist_idx_hbm, table_hbm, out_hbm):
        pltpu.sync_copy(hist_idx_hbm.at[pl.ds(base * H, BPW * H)], hidx_v)
        bufs = (hrows0_v, hrows1_v)

        def fire(b):
            buf = bufs[b % 2]
            c0 = pltpu.async_copy(
                table_hbm.at[hidx_v.at[pl.ds(b * H, H0)]],
                buf.at[pl.ds(0, H0)], sem)
            c1 = pltpu.async_copy(
                table_hbm.at[hidx_v.at[pl.ds(b * H + H0, H1)]],
                buf.at[pl.ds(H0, H1)], sem)
            return (c0, c1)

        inflight = fire(0)
        for b in range(BPW):
            for c in inflight:
                c.wait()
            buf = bufs[b % 2]
            if b + 1 < BPW:
                inflight = fire(b + 1)

            def body(j, acc):
                ea, ob = acc
                ea = tuple(ea[k] + buf[j, pl.ds(k * NLANE, NLANE)]
                           for k in range(DV))
                ob = tuple(ob[k] + buf[j + 1, pl.ds(k * NLANE, NLANE)]
                           for k in range(DV))
                return (ea, ob)

            zero = jnp.zeros((NLANE,), jnp.float32)
            init = (tuple(zero for _ in range(DV)),
                    tuple(zero for _ in range(DV)))
            ea, ob = plsc.parallel_loop(0, H, step=2, unroll=4,
                                        carry=init)(body)
            scale = jnp.float32(1.0 / H)
            for k in range(DV):
                pooled_v[b, pl.ds(k * NLANE, NLANE)] = (ea[k] + ob[k]) * scale
        pltpu.sync_copy(pooled_v, out_hbm.at[pl.ds(base, BPW)])

    pool_table(ur_idx_r, uremb_r, ure_out)
    pool_table(ir_idx_r, iremb_r, ire_out)


@functools.partial(
    pl.kernel,
    out_type=tuple(jax.ShapeDtypeStruct((B, D), jnp.float32) for _ in range(4)),
    mesh=plsc.VectorSubcoreMesh(core_axis_name="c", subcore_axis_name="s"),
    scratch_types=[
        pltpu.VMEM((BPW,), jnp.int32),
        pltpu.VMEM((BPW, D), jnp.float32),
        pltpu.VMEM((BPW * H,), jnp.int32),
        pltpu.VMEM((H, D), jnp.float32),
        pltpu.VMEM((H, D), jnp.float32),
        pltpu.VMEM((BPW, D), jnp.float32),
        pltpu.SemaphoreType.DMA,
    ],
    compiler_params=pltpu.CompilerParams(use_tc_tiling_on_sc=False),
)
def _sc_kernel(*refs):
    _sc_gather_pool(*refs)


def _tc_combine(ue_ref, ie_ref, ure_ref, ire_ref,
                wui_ref, wur_ref, wri_ref, wrr_ref,
                total_ref, regs_ref):
    ue = ue_ref[...]
    ie = ie_ref[...]
    ure = ure_ref[...]
    ire = ire_ref[...]
    inf_ui = jnp.dot(ue * ie, wui_ref[...], preferred_element_type=jnp.float32)
    inf_ur = jnp.dot(ue + ire, wur_ref[...], preferred_element_type=jnp.float32)
    inf_ri = jnp.dot(jnp.maximum(ure, ie), wri_ref[...],
                     preferred_element_type=jnp.float32)
    inf_rr = (jnp.dot(ure, wrr_ref[0:D, :], preferred_element_type=jnp.float32)
              + jnp.dot(ire, wrr_ref[D:2 * D, :],
                        preferred_element_type=jnp.float32))
    total_ref[...] = 0.25 * (inf_ui + inf_ur + inf_ri + inf_rr)
    regs = 0.001 * (jnp.sqrt(jnp.sum(ue * ue)) + jnp.sqrt(jnp.sum(ie * ie)))
    regs_ref[...] = regs.reshape(1, 1)


_tc_combine_call = pl.pallas_call(
    _tc_combine,
    out_shape=(jax.ShapeDtypeStruct((B, 1), jnp.float32),
               jax.ShapeDtypeStruct((1, 1), jnp.float32)),
)


def kernel(users, items, users_ratings, items_ratings,
           users_sparse_ratings, items_sparse_ratings,
           UsersEmb, ItemsEmb, UsersRatingsEmb, ItemsRatingsEmb,
           W_ui, W_ur, W_ri, W_rr):
    del users_sparse_ratings, items_sparse_ratings
    users = users.astype(jnp.int32)
    items = items.astype(jnp.int32)
    users_ratings = users_ratings.astype(jnp.int32).reshape(B * H)
    items_ratings = items_ratings.astype(jnp.int32).reshape(B * H)
    ue, ie, ure, ire = _sc_kernel(
        users, items, users_ratings, items_ratings,
        UsersEmb, ItemsEmb, UsersRatingsEmb, ItemsRatingsEmb)
    total, regs = _tc_combine_call(ue, ie, ure, ire, W_ui, W_ur, W_ri, W_rr)
    return total, regs[0, 0]


# speedup vs baseline: 4.8463x; 1.0159x over previous
import functools

import jax
import jax.numpy as jnp
from jax import lax
from jax.experimental import pallas as pl
from jax.experimental.pallas import tpu as pltpu
from jax.experimental.pallas import tpu_sc as plsc

B = 1024
D = 64
DP = 2 * D
H = 200
NC = 2
NS = 16
NW = NC * NS
BPW = B // NW
H0 = 104
H1 = H - H0
NLANE = 16
DV = D // NLANE


def _sc_gather_pool(users_r, items_r, ur_idx_r, ir_idx_r, big1_r, big2_r,
                    ue_out, ie_out, ure_out, ire_out,
                    idx_v, rows_v, hidx_v, hrows0_v, hrows1_v, pooled_v, sem):
    wid = lax.axis_index("s") * NC + lax.axis_index("c")
    base = wid * BPW

    def gather_simple(src_idx_hbm, out_hbm):
        pltpu.sync_copy(src_idx_hbm.at[pl.ds(base, BPW)], idx_v)
        pltpu.async_copy(big1_r.at[idx_v], rows_v, sem).wait()
        pltpu.sync_copy(rows_v, out_hbm.at[pl.ds(base, BPW)])

    gather_simple(users_r, ue_out)
    gather_simple(items_r, ie_out)

    def pool_table(hist_idx_hbm, out_hbm, lane_off):
        pltpu.sync_copy(hist_idx_hbm.at[pl.ds(base * H, BPW * H)], hidx_v)
        bufs = (hrows0_v, hrows1_v)

        def fire(b):
            buf = bufs[b % 2]
            c0 = pltpu.async_copy(
                big2_r.at[hidx_v.at[pl.ds(b * H, H0)]],
                buf.at[pl.ds(0, H0)], sem)
            c1 = pltpu.async_copy(
                big2_r.at[hidx_v.at[pl.ds(b * H + H0, H1)]],
                buf.at[pl.ds(H0, H1)], sem)
            return (c0, c1)

        inflight = fire(0)
        for b in range(BPW):
            for c in inflight:
                c.wait()
            buf = bufs[b % 2]
            if b + 1 < BPW:
                inflight = fire(b + 1)

            def body(j, acc):
                ea, ob = acc
                ea = tuple(ea[k] + buf[j, pl.ds(lane_off + k * NLANE, NLANE)]
                           for k in range(DV))
                ob = tuple(ob[k] + buf[j + 1, pl.ds(lane_off + k * NLANE, NLANE)]
                           for k in range(DV))
                return (ea, ob)

            zero = jnp.zeros((NLANE,), jnp.float32)
            init = (tuple(zero for _ in range(DV)),
                    tuple(zero for _ in range(DV)))
            ea, ob = plsc.parallel_loop(0, H, step=2, unroll=4,
                                        carry=init)(body)
            scale = jnp.float32(1.0 / H)
            for k in range(DV):
                pooled_v[b, pl.ds(lane_off + k * NLANE, NLANE)] = (
                    (ea[k] + ob[k]) * scale)
        pltpu.sync_copy(pooled_v, out_hbm.at[pl.ds(base, BPW)])

    pool_table(ur_idx_r, ure_out, 0)
    pool_table(ir_idx_r, ire_out, D)


@functools.partial(
    pl.kernel,
    out_type=tuple(jax.ShapeDtypeStruct((B, DP), jnp.float32)
                   for _ in range(4)),
    mesh=plsc.VectorSubcoreMesh(core_axis_name="c", subcore_axis_name="s"),
    scratch_types=[
        pltpu.VMEM((BPW,), jnp.int32),
        pltpu.VMEM((BPW, DP), jnp.float32),
        pltpu.VMEM((BPW * H,), jnp.int32),
        pltpu.VMEM((H, DP), jnp.float32),
        pltpu.VMEM((H, DP), jnp.float32),
        pltpu.VMEM((BPW, DP), jnp.float32),
        pltpu.SemaphoreType.DMA,
    ],
    compiler_params=pltpu.CompilerParams(use_tc_tiling_on_sc=True),
)
def _sc_kernel(*refs):
    _sc_gather_pool(*refs)


def _tc_combine(uei_ref, iei_ref, urp_ref, irp_ref,
                wui_ref, wur_ref, wri_ref, wrr_ref,
                total_ref, regs_ref):
    ue = uei_ref[:, 0:D]
    ie = iei_ref[:, D:DP]
    ure = urp_ref[:, 0:D]
    ire = irp_ref[:, D:DP]
    inf_ui = jnp.dot(ue * ie, wui_ref[...], preferred_element_type=jnp.float32)
    inf_ur = jnp.dot(ue + ire, wur_ref[...], preferred_element_type=jnp.float32)
    inf_ri = jnp.dot(jnp.maximum(ure, ie), wri_ref[...],
                     preferred_element_type=jnp.float32)
    inf_rr = (jnp.dot(ure, wrr_ref[0:D, :], preferred_element_type=jnp.float32)
              + jnp.dot(ire, wrr_ref[D:2 * D, :],
                        preferred_element_type=jnp.float32))
    total_ref[...] = 0.25 * (inf_ui + inf_ur + inf_ri + inf_rr)
    regs = 0.001 * (jnp.sqrt(jnp.sum(ue * ue)) + jnp.sqrt(jnp.sum(ie * ie)))
    regs_ref[...] = regs.reshape(1, 1)


_tc_combine_call = pl.pallas_call(
    _tc_combine,
    out_shape=(jax.ShapeDtypeStruct((B, 1), jnp.float32),
               jax.ShapeDtypeStruct((1, 1), jnp.float32)),
)


def kernel(users, items, users_ratings, items_ratings,
           users_sparse_ratings, items_sparse_ratings,
           UsersEmb, ItemsEmb, UsersRatingsEmb, ItemsRatingsEmb,
           W_ui, W_ur, W_ri, W_rr):
    del users_sparse_ratings, items_sparse_ratings
    users = users.astype(jnp.int32)
    items = items.astype(jnp.int32)
    users_ratings = users_ratings.astype(jnp.int32).reshape(B * H)
    items_ratings = items_ratings.astype(jnp.int32).reshape(B * H)
    big1 = jnp.concatenate([UsersEmb, ItemsEmb], axis=1)
    big2 = jnp.concatenate([UsersRatingsEmb, ItemsRatingsEmb], axis=1)
    uei, iei, urp, irp = _sc_kernel(
        users, items, users_ratings, items_ratings, big1, big2)
    total, regs = _tc_combine_call(uei, iei, urp, irp, W_ui, W_ur, W_ri, W_rr)
    return total, regs[0, 0]
